# baseline (speedup 1.0000x reference)
import functools

import jax
import jax.numpy as jnp
from jax import lax
from jax.experimental import pallas as pl
from jax.experimental.pallas import tpu as pltpu

NODE_DIM = 128
EDGE_DIM = 16
MAX_BOND = 8
ROWS_PER_STEP = 16


def _lut_kernel(am_s, am_v, bond2d, spec, table, ew, eb, nhi, nlo, pack):
    rows = []
    for j in range(64):
        m = j >> 3
        if m <= 2:
            rows.append(spec[m : m + 1, :])
        else:
            a = am_s[j]
            rows.append(table[pl.ds(a, 1), :])
    node64 = jnp.concatenate(rows, axis=0)
    hi = node64.astype(jnp.bfloat16)
    nhi[...] = hi
    nlo[...] = (node64 - hi.astype(jnp.float32)).astype(jnp.bfloat16)

    brows = []
    for j in range(64):
        a = am_s[j]
        brows.append(bond2d[pl.ds(a // MAX_BOND, 1), :])
    bond_rows = jnp.concatenate(brows, axis=0)
    lane8 = jax.lax.broadcasted_iota(jnp.int32, (64, MAX_BOND), 1)
    lsel = am_v[...] % MAX_BOND
    bc64 = jnp.sum(jnp.where(lane8 == lsel, bond_rows, 0), axis=1, keepdims=True)

    mask_ge = (lane8 >= bc64).astype(jnp.float32)
    s = jnp.dot(mask_ge, ew[...], preferred_element_type=jnp.float32)
    t = eb[...] - s
    thi = t.astype(jnp.bfloat16)
    pack[:, 0:EDGE_DIM] = thi
    pack[:, EDGE_DIM : 2 * EDGE_DIM] = (t - thi.astype(jnp.float32)).astype(jnp.bfloat16)
    pack[:, 2 * EDGE_DIM : 2 * EDGE_DIM + 1] = bc64.astype(jnp.bfloat16)


def _expand_kernel(idx_ref, nhi, nlo, pack, ewhi, ewlo, out_ref, *, rows, width):
    idxv = idx_ref[...]
    i0 = idxv[:, :, 0:1]
    i1 = idxv[:, :, 1:2]
    i2 = idxv[:, :, 2:3]
    c2 = i0 * MAX_BOND + i1
    q = lax.broadcasted_iota(jnp.int32, (rows, width, 64), 2)
    oh = (q == c2).astype(jnp.float32).astype(jnp.bfloat16)
    dn = (((2,), (0,)), ((), ()))
    node = lax.dot_general(
        oh, nhi[...], dn, preferred_element_type=jnp.float32
    ) + lax.dot_general(oh, nlo[...], dn, preferred_element_type=jnp.float32)
    r = lax.dot_general(oh, pack[...], dn, preferred_element_type=jnp.float32)
    base = r[:, :, 0:EDGE_DIM] + r[:, :, EDGE_DIM : 2 * EDGE_DIM]
    bci = r[:, :, 2 * EDGE_DIM : 2 * EDGE_DIM + 1].astype(jnp.int32)
    lane = lax.broadcasted_iota(jnp.int32, (rows, width, MAX_BOND), 2)
    two = jnp.where(i2 >= bci, 2.0, 1.0)
    oh8 = jnp.where(lane == i2, two, 0.0).astype(jnp.bfloat16)
    edge = (
        lax.dot_general(oh8, ewhi[...], dn, preferred_element_type=jnp.float32)
        + lax.dot_general(oh8, ewlo[...], dn, preferred_element_type=jnp.float32)
        + base
    )
    out_ref[:, :, :NODE_DIM] = node
    out_ref[:, :, NODE_DIM:] = edge


def kernel(idx, attached_motif_index_map, bonding_cnt, special_table, attached_table, edge_w, edge_b):
    lead_shape = idx.shape[:-1]
    nrows, width = lead_shape
    am64 = attached_motif_index_map[:MAX_BOND, :MAX_BOND].reshape(64)
    bond2d = bonding_cnt.reshape(-1, MAX_BOND)

    nhi, nlo, pack = pl.pallas_call(
        _lut_kernel,
        out_shape=(
            jax.ShapeDtypeStruct((64, NODE_DIM), jnp.bfloat16),
            jax.ShapeDtypeStruct((64, NODE_DIM), jnp.bfloat16),
            jax.ShapeDtypeStruct((64, 2 * EDGE_DIM + 1), jnp.bfloat16),
        ),
        in_specs=[
            pl.BlockSpec(memory_space=pltpu.SMEM),
            pl.BlockSpec(memory_space=pltpu.VMEM),
            pl.BlockSpec(memory_space=pltpu.VMEM),
            pl.BlockSpec(memory_space=pltpu.VMEM),
            pl.BlockSpec(memory_space=pltpu.VMEM),
            pl.BlockSpec(memory_space=pltpu.VMEM),
            pl.BlockSpec(memory_space=pltpu.VMEM),
        ],
    )(am64, am64.reshape(64, 1), bond2d, special_table, attached_table, edge_w, edge_b.reshape(1, EDGE_DIM))

    ewhi = edge_w.astype(jnp.bfloat16)
    ewlo = (edge_w - ewhi.astype(jnp.float32)).astype(jnp.bfloat16)

    g = ROWS_PER_STEP
    assert nrows % g == 0

    out = pl.pallas_call(
        functools.partial(_expand_kernel, rows=g, width=width),
        grid=(nrows // g,),
        out_shape=jax.ShapeDtypeStruct((nrows, width, NODE_DIM + EDGE_DIM), jnp.float32),
        in_specs=[
            pl.BlockSpec((g, width, 3), lambda i: (i, 0, 0)),
            pl.BlockSpec((64, NODE_DIM), lambda i: (0, 0)),
            pl.BlockSpec((64, NODE_DIM), lambda i: (0, 0)),
            pl.BlockSpec((64, 2 * EDGE_DIM + 1), lambda i: (0, 0)),
            pl.BlockSpec((MAX_BOND, EDGE_DIM), lambda i: (0, 0)),
            pl.BlockSpec((MAX_BOND, EDGE_DIM), lambda i: (0, 0)),
        ],
        out_specs=pl.BlockSpec((g, width, NODE_DIM + EDGE_DIM), lambda i: (i, 0, 0)),
        compiler_params=pltpu.CompilerParams(dimension_semantics=("parallel",)),
    )(idx, nhi, nlo, pack, ewhi, ewlo)

    return out


# speedup vs baseline: 1.3394x; 1.3394x over previous
import functools

import jax
import jax.numpy as jnp
from jax import lax
from jax.experimental import pallas as pl
from jax.experimental.pallas import tpu as pltpu

NODE_DIM = 128
EDGE_DIM = 16
OUT_DIM = NODE_DIM + EDGE_DIM
MAX_BOND = 8
NCOMBO = 512
ROWS_PER_STEP = 16


def _lut_kernel(am_s, am_v, bond2d, spec, table, ew, eb, luthi, lutlo):
    rows = []
    for j in range(64):
        m = j >> 3
        if m <= 2:
            rows.append(spec[m : m + 1, :])
        else:
            a = am_s[j]
            rows.append(table[pl.ds(a, 1), :])
    node64 = jnp.concatenate(rows, axis=0)

    brows = []
    for j in range(64):
        a = am_s[j]
        brows.append(bond2d[pl.ds(a // MAX_BOND, 1), :])
    bond_rows = jnp.concatenate(brows, axis=0)
    lane8 = lax.broadcasted_iota(jnp.int32, (64, MAX_BOND), 1)
    lsel = am_v[...] % MAX_BOND
    bc64 = jnp.sum(jnp.where(lane8 == lsel, bond_rows, 0), axis=1, keepdims=True)

    r512 = lax.broadcasted_iota(jnp.int32, (NCOMBO, 64), 0)
    q64 = lax.broadcasted_iota(jnp.int32, (NCOMBO, 64), 1)
    ohe = (r512 // MAX_BOND == q64).astype(jnp.float32)
    node512 = jnp.dot(ohe, node64, preferred_element_type=jnp.float32)
    bc512 = jnp.dot(ohe, bc64.astype(jnp.float32), preferred_element_type=jnp.float32)
    bc512 = bc512.astype(jnp.int32)

    rowid = lax.broadcasted_iota(jnp.int32, (NCOMBO, MAX_BOND), 0)
    lane = lax.broadcasted_iota(jnp.int32, (NCOMBO, MAX_BOND), 1)
    bpos = rowid % MAX_BOND
    one_hot = jnp.where(lane == bpos, 1.0, jnp.where(lane < bc512, 0.0, -1.0))
    edge512 = jnp.dot(one_hot, ew[...], preferred_element_type=jnp.float32) + eb[...]

    nh = node512.astype(jnp.bfloat16)
    luthi[:, :NODE_DIM] = nh
    lutlo[:, :NODE_DIM] = (node512 - nh.astype(jnp.float32)).astype(jnp.bfloat16)
    eh = edge512.astype(jnp.bfloat16)
    luthi[:, NODE_DIM:] = eh
    lutlo[:, NODE_DIM:] = (edge512 - eh.astype(jnp.float32)).astype(jnp.bfloat16)


def _expand_kernel(idx_ref, luthi, lutlo, out_ref, *, rows, width):
    idxv = idx_ref[...]
    c = (idxv[:, :, 0:1] * MAX_BOND + idxv[:, :, 1:2]) * MAX_BOND + idxv[:, :, 2:3]
    q = lax.broadcasted_iota(jnp.int32, (rows, width, NCOMBO), 2)
    oh = (q == c).astype(jnp.float32).astype(jnp.bfloat16)
    dn = (((2,), (0,)), ((), ()))
    out_ref[...] = lax.dot_general(
        oh, luthi[...], dn, preferred_element_type=jnp.float32
    ) + lax.dot_general(oh, lutlo[...], dn, preferred_element_type=jnp.float32)


def kernel(idx, attached_motif_index_map, bonding_cnt, special_table, attached_table, edge_w, edge_b):
    nrows, width = idx.shape[:-1]
    am64 = attached_motif_index_map[:MAX_BOND, :MAX_BOND].reshape(64)
    bond2d = bonding_cnt.reshape(-1, MAX_BOND)

    luthi, lutlo = pl.pallas_call(
        _lut_kernel,
        out_shape=(
            jax.ShapeDtypeStruct((NCOMBO, OUT_DIM), jnp.bfloat16),
            jax.ShapeDtypeStruct((NCOMBO, OUT_DIM), jnp.bfloat16),
        ),
        in_specs=[
            pl.BlockSpec(memory_space=pltpu.SMEM),
            pl.BlockSpec(memory_space=pltpu.VMEM),
            pl.BlockSpec(memory_space=pltpu.VMEM),
            pl.BlockSpec(memory_space=pltpu.VMEM),
            pl.BlockSpec(memory_space=pltpu.VMEM),
            pl.BlockSpec(memory_space=pltpu.VMEM),
            pl.BlockSpec(memory_space=pltpu.VMEM),
        ],
    )(am64, am64.reshape(64, 1), bond2d, special_table, attached_table, edge_w, edge_b.reshape(1, EDGE_DIM))

    g = ROWS_PER_STEP
    assert nrows % g == 0

    out = pl.pallas_call(
        functools.partial(_expand_kernel, rows=g, width=width),
        grid=(nrows // g,),
        out_shape=jax.ShapeDtypeStruct((nrows, width, OUT_DIM), jnp.float32),
        in_specs=[
            pl.BlockSpec((g, width, 3), lambda i: (i, 0, 0)),
            pl.BlockSpec((NCOMBO, OUT_DIM), lambda i: (0, 0)),
            pl.BlockSpec((NCOMBO, OUT_DIM), lambda i: (0, 0)),
        ],
        out_specs=pl.BlockSpec((g, width, OUT_DIM), lambda i: (i, 0, 0)),
        compiler_params=pltpu.CompilerParams(dimension_semantics=("parallel",)),
    )(idx, luthi, lutlo)

    return out


# speedup vs baseline: 8.9509x; 6.6827x over previous
import functools

import jax
import jax.numpy as jnp
from jax import lax
from jax.experimental import pallas as pl
from jax.experimental.pallas import tpu as pltpu

NODE_DIM = 128
EDGE_DIM = 16
OUT_DIM = NODE_DIM + EDGE_DIM
MAX_BOND = 8
NCOMBO = 512
LANES_PER_STEP = 256


def _lut_kernel(am_s, am_v, bond2d, spec, table, ew, eb, luthi, lutlo):
    rows = []
    for j in range(64):
        m = j >> 3
        if m <= 2:
            rows.append(spec[m : m + 1, :])
        else:
            a = am_s[j]
            rows.append(table[pl.ds(a, 1), :])
    node64 = jnp.concatenate(rows, axis=0)

    brows = []
    for j in range(64):
        a = am_s[j]
        brows.append(bond2d[pl.ds(a // MAX_BOND, 1), :])
    bond_rows = jnp.concatenate(brows, axis=0)
    lane8 = lax.broadcasted_iota(jnp.int32, (64, MAX_BOND), 1)
    lsel = am_v[...] % MAX_BOND
    bc64 = jnp.sum(jnp.where(lane8 == lsel, bond_rows, 0), axis=1, keepdims=True)

    r512 = lax.broadcasted_iota(jnp.int32, (NCOMBO, 64), 0)
    q64 = lax.broadcasted_iota(jnp.int32, (NCOMBO, 64), 1)
    ohe = (r512 // MAX_BOND == q64).astype(jnp.float32)
    node512 = jnp.dot(ohe, node64, preferred_element_type=jnp.float32)
    bc512 = jnp.dot(ohe, bc64.astype(jnp.float32), preferred_element_type=jnp.float32)
    bc512 = bc512.astype(jnp.int32)

    rowid = lax.broadcasted_iota(jnp.int32, (NCOMBO, MAX_BOND), 0)
    lane = lax.broadcasted_iota(jnp.int32, (NCOMBO, MAX_BOND), 1)
    bpos = rowid % MAX_BOND
    one_hot = jnp.where(lane == bpos, 1.0, jnp.where(lane < bc512, 0.0, -1.0))
    edge512 = jnp.dot(one_hot, ew[...], preferred_element_type=jnp.float32) + eb[...]

    nt = node512.T
    et = edge512.T
    nh = nt.astype(jnp.bfloat16)
    luthi[:NODE_DIM, :] = nh
    lutlo[:NODE_DIM, :] = (nt - nh.astype(jnp.float32)).astype(jnp.bfloat16)
    eh = et.astype(jnp.bfloat16)
    luthi[NODE_DIM:, :] = eh
    lutlo[NODE_DIM:, :] = (et - eh.astype(jnp.float32)).astype(jnp.bfloat16)


def _expand_kernel(idxt_ref, luthi, lutlo, out_ref, *, width, lanes):
    m = idxt_ref[0, :, :]
    a = idxt_ref[1, :, :]
    b = idxt_ref[2, :, :]
    c = (m * MAX_BOND + a) * MAX_BOND + b
    si = lax.broadcasted_iota(jnp.int32, (NCOMBO, lanes), 0)
    hi = luthi[...]
    lo = lutlo[...]
    for w in range(width):
        cw = c[w : w + 1, :]
        oh = (si == cw).astype(jnp.float32).astype(jnp.bfloat16)
        out_ref[w] = jnp.dot(hi, oh, preferred_element_type=jnp.float32) + jnp.dot(
            lo, oh, preferred_element_type=jnp.float32
        )


def kernel(idx, attached_motif_index_map, bonding_cnt, special_table, attached_table, edge_w, edge_b):
    nrows, width = idx.shape[:-1]
    am64 = attached_motif_index_map[:MAX_BOND, :MAX_BOND].reshape(64)
    bond2d = bonding_cnt.reshape(-1, MAX_BOND)

    luthi, lutlo = pl.pallas_call(
        _lut_kernel,
        out_shape=(
            jax.ShapeDtypeStruct((OUT_DIM, NCOMBO), jnp.bfloat16),
            jax.ShapeDtypeStruct((OUT_DIM, NCOMBO), jnp.bfloat16),
        ),
        in_specs=[
            pl.BlockSpec(memory_space=pltpu.SMEM),
            pl.BlockSpec(memory_space=pltpu.VMEM),
            pl.BlockSpec(memory_space=pltpu.VMEM),
            pl.BlockSpec(memory_space=pltpu.VMEM),
            pl.BlockSpec(memory_space=pltpu.VMEM),
            pl.BlockSpec(memory_space=pltpu.VMEM),
            pl.BlockSpec(memory_space=pltpu.VMEM),
        ],
    )(am64, am64.reshape(64, 1), bond2d, special_table, attached_table, edge_w, edge_b.reshape(1, EDGE_DIM))

    lanes = LANES_PER_STEP
    assert nrows % lanes == 0

    idxt = jnp.transpose(idx, (2, 1, 0))

    outt = pl.pallas_call(
        functools.partial(_expand_kernel, width=width, lanes=lanes),
        grid=(nrows // lanes,),
        out_shape=jax.ShapeDtypeStruct((width, OUT_DIM, nrows), jnp.float32),
        in_specs=[
            pl.BlockSpec((3, width, lanes), lambda i: (0, 0, i)),
            pl.BlockSpec((OUT_DIM, NCOMBO), lambda i: (0, 0)),
            pl.BlockSpec((OUT_DIM, NCOMBO), lambda i: (0, 0)),
        ],
        out_specs=pl.BlockSpec((width, OUT_DIM, lanes), lambda i: (0, 0, i)),
        compiler_params=pltpu.CompilerParams(dimension_semantics=("parallel",)),
    )(idxt, luthi, lutlo)

    return jnp.transpose(outt, (2, 0, 1))
